# baseline (speedup 1.0000x reference)
import functools

import jax
import jax.numpy as jnp
from jax import lax
from jax.experimental import pallas as pl
from jax.experimental.pallas import tpu as pltpu
from jax.experimental.pallas import tpu_sc as plsc

D_MODEL = 64
D_PAD = 128
PAD_IDX = 0

NUM_CORES = 2
NUM_SUBCORES = 16
LANES = 16
NUM_WORKERS = NUM_CORES * NUM_SUBCORES

ROWS_PER_GATHER = 128
GROUPS_PER_GATHER = ROWS_PER_GATHER // LANES


def _make_embed(total_rows: int, vocab: int):
    chunks_per_w = total_rows // (NUM_WORKERS * ROWS_PER_GATHER)
    rows_per_w = chunks_per_w * ROWS_PER_GATHER
    assert chunks_per_w * NUM_WORKERS * ROWS_PER_GATHER == total_rows
    assert chunks_per_w % 2 == 0

    mesh = plsc.VectorSubcoreMesh(core_axis_name="c", subcore_axis_name="s")

    @functools.partial(
        pl.kernel,
        out_type=jax.ShapeDtypeStruct((total_rows, D_MODEL), jnp.float32),
        mesh=mesh,
        compiler_params=pltpu.CompilerParams(use_tc_tiling_on_sc=False),
        scratch_types=[
            pltpu.VMEM((chunks_per_w, ROWS_PER_GATHER), jnp.int32),
            pltpu.VMEM((ROWS_PER_GATHER, D_MODEL), jnp.float32),
            pltpu.VMEM((ROWS_PER_GATHER, D_MODEL), jnp.float32),
            pltpu.SemaphoreType.DMA,
            pltpu.SemaphoreType.DMA,
            pltpu.SemaphoreType.DMA,
            pltpu.SemaphoreType.DMA,
        ],
    )
    def embed(x_hbm, table_hbm, out_hbm, idx_v, rows0, rows1, sg0, sg1, so0, so1):
        wid = lax.axis_index("s") * NUM_CORES + lax.axis_index("c")
        wbase = wid * chunks_per_w
        rbase = wid * rows_per_w

        pltpu.sync_copy(x_hbm.at[pl.ds(wbase, chunks_per_w)], idx_v)

        rows = (rows0, rows1)
        sg = (sg0, sg1)
        so = (so0, so1)

        def gather(j, b):
            return pltpu.make_async_copy(table_hbm.at[idx_v.at[j]], rows[b], sg[b])

        def write(j, b):
            return pltpu.make_async_copy(
                rows[b],
                out_hbm.at[pl.ds(rbase + j * ROWS_PER_GATHER, ROWS_PER_GATHER)],
                so[b],
            )

        def fixup(j, b):
            rb = rows[b]

            one = jnp.ones((LANES,), jnp.int32)
            zero = jnp.zeros((LANES,), jnp.int32)
            acc = jnp.where(idx_v[j, pl.ds(0, LANES)] == PAD_IDX, one, zero)
            for g in range(1, GROUPS_PER_GATHER):
                acc = acc | jnp.where(
                    idx_v[j, pl.ds(g * LANES, LANES)] == PAD_IDX, one, zero
                )
            has_pad = acc[0]
            for i in range(1, LANES):
                has_pad = has_pad | acc[i]

            @pl.when(has_pad > 0)
            def _():
                zeros = jnp.zeros((LANES,), jnp.float32)

                def grp(g, carry):
                    vec = idx_v[j, pl.ds(g * LANES, LANES)]
                    for r in range(LANES):
                        @pl.when(vec[r] == PAD_IDX)
                        def _():
                            row = g * LANES + r
                            for c in range(D_MODEL // LANES):
                                rb[row, pl.ds(c * LANES, LANES)] = zeros

                    return carry

                lax.fori_loop(0, GROUPS_PER_GATHER, grp, 0)

        gather(0, 0).start()

        def step(j, b):
            bn = 1 - b

            @pl.when(j + 1 < chunks_per_w)
            def _():
                @pl.when(j >= 1)
                def _():
                    write(j - 1, bn).wait()

                gather(j + 1, bn).start()

            gather(j, b).wait()
            fixup(j, b)
            write(j, b).start()

        def outer(jo, carry):
            step(2 * jo, 0)
            step(2 * jo + 1, 1)
            return carry

        lax.fori_loop(0, chunks_per_w // 2, outer, 0)
        write(chunks_per_w - 2, 0).wait()
        write(chunks_per_w - 1, 1).wait()

    return embed


def kernel(x, table):
    batch, seq = x.shape
    total = batch * seq
    vocab = table.shape[0]
    x2 = x.reshape(total // ROWS_PER_GATHER, ROWS_PER_GATHER).astype(jnp.int32)
    out = _make_embed(total, vocab)(x2, table)
    return out.reshape(batch, seq, D_MODEL)


# speedup vs baseline: 1.3215x; 1.3215x over previous
import functools

import jax
import jax.numpy as jnp
from jax import lax
from jax.experimental import pallas as pl
from jax.experimental.pallas import tpu as pltpu
from jax.experimental.pallas import tpu_sc as plsc

D_MODEL = 64
D_PAD = 128
PAD_IDX = 0

NUM_CORES = 2
NUM_SUBCORES = 16
LANES = 16
NUM_WORKERS = NUM_CORES * NUM_SUBCORES

ROWS_PER_GATHER = 128
GROUPS_PER_GATHER = ROWS_PER_GATHER // LANES


def _make_embed(total_rows: int, vocab: int):
    chunks_per_w = total_rows // (NUM_WORKERS * ROWS_PER_GATHER)
    rows_per_w = chunks_per_w * ROWS_PER_GATHER
    assert chunks_per_w * NUM_WORKERS * ROWS_PER_GATHER == total_rows
    assert chunks_per_w % 2 == 0

    mesh = plsc.VectorSubcoreMesh(core_axis_name="c", subcore_axis_name="s")

    @functools.partial(
        pl.kernel,
        out_type=jax.ShapeDtypeStruct((total_rows, D_PAD), jnp.float32),
        mesh=mesh,
        compiler_params=pltpu.CompilerParams(use_tc_tiling_on_sc=False),
        scratch_types=[
            pltpu.VMEM((chunks_per_w, ROWS_PER_GATHER), jnp.int32),
            pltpu.VMEM((ROWS_PER_GATHER, D_MODEL), jnp.float32),
            pltpu.VMEM((ROWS_PER_GATHER, D_MODEL), jnp.float32),
            pltpu.SemaphoreType.DMA,
            pltpu.SemaphoreType.DMA,
            pltpu.SemaphoreType.DMA,
            pltpu.SemaphoreType.DMA,
        ],
    )
    def embed(x_hbm, table_hbm, out_hbm, idx_v, rows0, rows1, sg0, sg1, so0, so1):
        wid = lax.axis_index("s") * NUM_CORES + lax.axis_index("c")
        wbase = wid * chunks_per_w
        rbase = wid * rows_per_w

        pltpu.sync_copy(x_hbm.at[pl.ds(wbase, chunks_per_w)], idx_v)

        rows = (rows0, rows1)
        sg = (sg0, sg1)
        so = (so0, so1)

        def gather(j, b):
            return pltpu.make_async_copy(table_hbm.at[idx_v.at[j]], rows[b], sg[b])

        def write(j, b):
            return pltpu.make_async_copy(
                rows[b],
                out_hbm.at[
                    pl.ds(rbase + j * ROWS_PER_GATHER, ROWS_PER_GATHER),
                    pl.ds(0, D_MODEL),
                ],
                so[b],
            )

        def fixup(j, b):
            rb = rows[b]

            one = jnp.ones((LANES,), jnp.int32)
            zero = jnp.zeros((LANES,), jnp.int32)
            acc = jnp.where(idx_v[j, pl.ds(0, LANES)] == PAD_IDX, one, zero)
            for g in range(1, GROUPS_PER_GATHER):
                acc = acc | jnp.where(
                    idx_v[j, pl.ds(g * LANES, LANES)] == PAD_IDX, one, zero
                )
            has_pad = acc[0]
            for i in range(1, LANES):
                has_pad = has_pad | acc[i]

            @pl.when(has_pad > 0)
            def _():
                zeros = jnp.zeros((LANES,), jnp.float32)

                def grp(g, carry):
                    vec = idx_v[j, pl.ds(g * LANES, LANES)]
                    for r in range(LANES):
                        @pl.when(vec[r] == PAD_IDX)
                        def _():
                            row = g * LANES + r
                            for c in range(D_MODEL // LANES):
                                rb[row, pl.ds(c * LANES, LANES)] = zeros

                    return carry

                lax.fori_loop(0, GROUPS_PER_GATHER, grp, 0)

        gather(0, 0).start()

        def step(j, b):
            bn = 1 - b

            @pl.when(j + 1 < chunks_per_w)
            def _():
                @pl.when(j >= 1)
                def _():
                    write(j - 1, bn).wait()

                gather(j + 1, bn).start()

            gather(j, b).wait()
            fixup(j, b)
            write(j, b).start()

        def outer(jo, carry):
            step(2 * jo, 0)
            step(2 * jo + 1, 1)
            return carry

        lax.fori_loop(0, chunks_per_w // 2, outer, 0)
        write(chunks_per_w - 2, 0).wait()
        write(chunks_per_w - 1, 1).wait()

    return embed


def kernel(x, table):
    batch, seq = x.shape
    total = batch * seq
    vocab = table.shape[0]
    from jax.experimental.layout import Format, Layout
    from jax.experimental.layout import with_layout_constraint

    x2 = x.reshape(total // ROWS_PER_GATHER, ROWS_PER_GATHER).astype(jnp.int32)
    out = _make_embed(total, vocab)(x2, table)
    return out[:, :D_MODEL].reshape(batch, seq, D_MODEL)


# speedup vs baseline: 1.4214x; 1.0756x over previous
import functools

import jax
import jax.numpy as jnp
from jax import lax
from jax.experimental import pallas as pl
from jax.experimental.pallas import tpu as pltpu
from jax.experimental.pallas import tpu_sc as plsc

D_MODEL = 64
D_PAD = 128
PAD_IDX = 0

NUM_CORES = 2
NUM_SUBCORES = 16
LANES = 16
NUM_WORKERS = NUM_CORES * NUM_SUBCORES

ROWS_PER_GATHER = 128
GROUPS_PER_GATHER = ROWS_PER_GATHER // LANES


def _make_embed(total_rows: int, vocab: int):
    chunks_per_w = total_rows // (NUM_WORKERS * ROWS_PER_GATHER)
    rows_per_w = chunks_per_w * ROWS_PER_GATHER
    assert chunks_per_w * NUM_WORKERS * ROWS_PER_GATHER == total_rows
    assert chunks_per_w % 2 == 0

    mesh = plsc.VectorSubcoreMesh(core_axis_name="c", subcore_axis_name="s")

    @functools.partial(
        pl.kernel,
        out_type=jax.ShapeDtypeStruct((total_rows, D_PAD), jnp.float32),
        mesh=mesh,
        compiler_params=pltpu.CompilerParams(use_tc_tiling_on_sc=False),
        scratch_types=[
            pltpu.VMEM((chunks_per_w, ROWS_PER_GATHER), jnp.int32),
            pltpu.VMEM((ROWS_PER_GATHER, D_MODEL), jnp.float32),
            pltpu.VMEM((ROWS_PER_GATHER, D_MODEL), jnp.float32),
            pltpu.SemaphoreType.DMA,
            pltpu.SemaphoreType.DMA,
            pltpu.SemaphoreType.DMA,
            pltpu.SemaphoreType.DMA,
        ],
    )
    def embed(x_hbm, table_hbm, out_hbm, idx_v, rows0, rows1, sg0, sg1, so0, so1):
        wid = lax.axis_index("s") * NUM_CORES + lax.axis_index("c")
        wbase = wid * chunks_per_w
        rbase = wid * rows_per_w

        pltpu.sync_copy(x_hbm.at[pl.ds(wbase, chunks_per_w)], idx_v)

        rows = (rows0, rows1)
        sg = (sg0, sg1)
        so = (so0, so1)

        def gather(j, b):
            return pltpu.make_async_copy(table_hbm.at[idx_v.at[j]], rows[b], sg[b])

        def write(j, b):
            return pltpu.make_async_copy(
                rows[b],
                out_hbm.at[
                    pl.ds(rbase + j * ROWS_PER_GATHER, ROWS_PER_GATHER),
                    pl.ds(0, D_MODEL),
                ],
                so[b],
            )

        def fixup(j, b):
            rb = rows[b]

            one = jnp.ones((LANES,), jnp.int32)
            zero = jnp.zeros((LANES,), jnp.int32)
            acc = jnp.where(idx_v[j, pl.ds(0, LANES)] == PAD_IDX, one, zero)
            for g in range(1, GROUPS_PER_GATHER):
                acc = acc | jnp.where(
                    idx_v[j, pl.ds(g * LANES, LANES)] == PAD_IDX, one, zero
                )
            has_pad = acc[0]
            for i in range(1, LANES):
                has_pad = has_pad | acc[i]

            @pl.when(has_pad > 0)
            def _():
                zeros = jnp.zeros((LANES,), jnp.float32)

                def grp(g, carry):
                    vec = idx_v[j, pl.ds(g * LANES, LANES)]
                    for r in range(LANES):
                        @pl.when(vec[r] == PAD_IDX)
                        def _():
                            row = g * LANES + r
                            for c in range(D_MODEL // LANES):
                                rb[row, pl.ds(c * LANES, LANES)] = zeros

                    return carry

                lax.fori_loop(0, GROUPS_PER_GATHER, grp, 0)

        gather(0, 0).start()

        def step(j, b):
            bn = 1 - b

            @pl.when(j + 1 < chunks_per_w)
            def _():
                @pl.when(j >= 1)
                def _():
                    write(j - 1, bn).wait()

                gather(j + 1, bn).start()

            gather(j, b).wait()
            fixup(j, b)
            write(j, b).start()

        def outer(jo, carry):
            step(2 * jo, 0)
            step(2 * jo + 1, 1)
            return carry

        lax.fori_loop(0, chunks_per_w // 2, outer, 0)
        write(chunks_per_w - 2, 0).wait()
        write(chunks_per_w - 1, 1).wait()

    return embed


def kernel(x, table):
    batch, seq = x.shape
    total = batch * seq
    vocab = table.shape[0]
    from jax.experimental.layout import Format, Layout
    from jax.experimental.layout import with_layout_constraint

    table_p = jax.lax.dynamic_update_slice(
        jnp.zeros((vocab, D_PAD), jnp.float32), table, (0, 0)
    )
    table_v = table_p.reshape(2 * vocab, D_MODEL)
    x2 = (x.reshape(total // ROWS_PER_GATHER, ROWS_PER_GATHER) * 2).astype(jnp.int32)
    out = _make_embed(total, vocab)(x2, table_v)
    return out[:, :D_MODEL].reshape(batch, seq, D_MODEL)
